# baseline (speedup 1.0000x reference)
import functools

import jax
import jax.numpy as jnp
from jax import lax
from jax.experimental import pallas as pl
from jax.experimental.pallas import tpu as pltpu
from jax.experimental.pallas import tpu_sc as plsc

N = 10000
E = 160000
NR = 10240
F = 8
NSC = 2
NTPC = 16
NTILES = NSC * NTPC
CHUNK = 128
EPT0 = E // NTPC
NCHT = E // CHUNK
NCHB = NCHT // NTILES
NXTRA = NCHT - NCHB * NTILES
HALF = NR // NSC
RPT = HALF // NTPC
ZPT = NR // NTPC
NRR = NR * F // 128
RING = 16
LOOK = 8
SQ = 8

_f32 = jnp.float32
_i32 = jnp.int32


def _rsqrt_newton(d):
    i = plsc.bitcast(d, _i32)
    y = plsc.bitcast(0x5F3759DF - (i >> 1), _f32)
    y = y * (1.5 - 0.5 * d * y * y)
    y = y * (1.5 - 0.5 * d * y * y)
    y = y * (1.5 - 0.5 * d * y * y)
    return y


def _agg_stream(u_hbm, srcv, dstv, bufs, semg, sems, shared, nch):
    @pl.loop(0, LOOK)
    def _prime(j):
        pltpu.async_copy(u_hbm.at[srcv.at[j]], bufs.at[j], semg)

    @pl.loop(0, nch)
    def _main(j):
        @pl.when(j + LOOK < nch)
        def _():
            jn = j + LOOK
            b = lax.rem(jn, RING)

            @pl.when(jn >= RING)
            def _():
                pltpu.make_async_copy(bufs.at[b], shared.at[dstv.at[jn - RING]],
                                      sems).wait()

            pltpu.async_copy(u_hbm.at[srcv.at[jn]], bufs.at[b], semg)

        bj = lax.rem(j, RING)
        pltpu.make_async_copy(u_hbm.at[srcv.at[j]], bufs.at[bj], semg).wait()
        pltpu.async_copy(bufs.at[bj], shared.at[dstv.at[j]], sems, add=True)

    @pl.loop(nch - RING, nch)
    def _drain(j):
        pltpu.make_async_copy(bufs.at[lax.rem(j, RING)], shared.at[dstv.at[j]],
                              sems).wait()


def _s01_body(e_hbm, pt_hbm, eta_hbm, phi_hbm, m_hbm,
              src_hbm, dst_hbm, zeros_hbm,
              u1a_hbm, u1b_hbm, disa_hbm, disb_hbm, d1_hbm,
              dstv1, histv, hpart, colb, ubf, disbf, disloc,
              srcv, dstv, bufs, semg, sems, hists, shared):
    c = lax.axis_index("c")
    s = lax.axis_index("s")
    wid = c * NTPC + s
    zrow = s * ZPT

    @pl.loop(0, NR // 16)
    def _zero(j):
        histv[pl.ds(j * 16, 16)] = jnp.zeros((16,), _i32)

    pltpu.sync_copy(zeros_hbm.at[pl.ds(zrow, ZPT)], shared.at[pl.ds(zrow, ZPT)])
    ebase = wid * NCHB + jnp.minimum(wid, NXTRA)
    nch = NCHB + (wid < NXTRA).astype(_i32)

    @pl.when(wid < NXTRA)
    def _():
        pltpu.sync_copy(src_hbm.at[pl.ds(ebase, NCHB + 1)], srcv)
        pltpu.sync_copy(dst_hbm.at[pl.ds(ebase, NCHB + 1)], dstv)

    @pl.when(wid >= NXTRA)
    def _():
        pltpu.sync_copy(src_hbm.at[pl.ds(ebase, NCHB)], srcv.at[pl.ds(0, NCHB)])
        pltpu.sync_copy(dst_hbm.at[pl.ds(ebase, NCHB)], dstv.at[pl.ds(0, NCHB)])

    hbase = s * (NCHT // NTPC) + jnp.minimum(s, NCHT % NTPC)
    nchh = NCHT // NTPC + (s < NCHT % NTPC).astype(_i32)

    @pl.when(s < NCHT % NTPC)
    def _():
        pltpu.sync_copy(dst_hbm.at[pl.ds(hbase, NCHT // NTPC + 1)], dstv1)

    @pl.when(s >= NCHT % NTPC)
    def _():
        pltpu.sync_copy(dst_hbm.at[pl.ds(hbase, NCHT // NTPC)],
                        dstv1.at[pl.ds(0, NCHT // NTPC)])

    @pl.loop(0, nchh)
    def _count(j):
        for k in range(CHUNK // 16):
            dv = dstv1[j, pl.ds(k * 16, 16)]
            cnt, last = plsc.scan_count(dv)
            plsc.addupdate_scatter(histv, [dv], cnt, mask=last)

    pltpu.sync_copy(histv, hists.at[s])

    @pl.loop(0, 5 * ZPT // 16)
    def _zc(j):
        colb[pl.ds(j * 16, 16)] = jnp.zeros((16,), _f32)

    base = s * ZPT
    feats = (e_hbm, pt_hbm, eta_hbm, phi_hbm, m_hbm)

    @pl.when(base + ZPT <= N)
    def _():
        for f, fh in enumerate(feats):
            pltpu.sync_copy(fh.at[pl.ds(base, ZPT)], colb.at[pl.ds(f * ZPT, ZPT)])

    @pl.when(base + ZPT > N)
    def _():
        for f, fh in enumerate(feats):
            pltpu.sync_copy(fh.at[pl.ds(base, N - 15 * ZPT)],
                            colb.at[pl.ds(f * ZPT, N - 15 * ZPT)])

    plsc.subcore_barrier()

    for t in range(NTPC):
        pltpu.sync_copy(hists.at[t, pl.ds(base, ZPT)], hpart.at[t])

    @pl.loop(0, ZPT // 16)
    def _deg(j):
        tot = hpart[0, pl.ds(j * 16, 16)]
        for t in range(1, NTPC):
            tot = tot + hpart[t, pl.ds(j * 16, 16)]
        d = tot.astype(_f32) + 1.0
        disloc[pl.ds(j * 16, 16)] = _rsqrt_newton(d)

    iota = lax.iota(_i32, 16)

    @pl.loop(0, ZPT * F // 32, unroll=2)
    def _scale(j):
        for h in range(2):
            jj = 2 * j + h
            g = 16 * jj + iota
            r = g >> 3
            f = g & 7
            xv = plsc.load_gather(colb, [jnp.minimum(f * ZPT + r, 5 * ZPT - 1)])
            xv = jnp.where(f < 5, xv, 0.0)
            dv = plsc.load_gather(disloc, [r])
            plsc.store_scatter(disbf, [r, f], dv)
            plsc.store_scatter(ubf, [r, f], dv * xv)

    @pl.when(c == 0)
    def _():
        pltpu.sync_copy(disbf, disa_hbm.at[pl.ds(base, ZPT)])
        pltpu.sync_copy(ubf, u1a_hbm.at[pl.ds(base, ZPT)])

    @pl.when(c == 1)
    def _():
        pltpu.sync_copy(disbf, disb_hbm.at[pl.ds(base, ZPT)])
        pltpu.sync_copy(ubf, u1b_hbm.at[pl.ds(base, ZPT)])

    plsc.subcore_barrier()

    @pl.when(c == 0)
    def _():
        _agg_stream(u1a_hbm, srcv, dstv, bufs, semg, sems, shared, nch)

    @pl.when(c == 1)
    def _():
        _agg_stream(u1b_hbm, srcv, dstv, bufs, semg, sems, shared, nch)

    plsc.subcore_barrier()
    pltpu.sync_copy(shared.at[pl.ds(zrow, ZPT)], d1_hbm.at[c, pl.ds(zrow, ZPT)])


def _agg_body(u_hbm, src_hbm, dst_hbm, zeros_hbm, out_hbm,
              srcv, dstv, bufs, semg, sems, shared):
    c = lax.axis_index("c")
    s = lax.axis_index("s")
    wid = c * NTPC + s
    zrow = s * ZPT
    pltpu.sync_copy(zeros_hbm.at[pl.ds(zrow, ZPT)], shared.at[pl.ds(zrow, ZPT)])

    base = wid * NCHB + jnp.minimum(wid, NXTRA)
    nch = NCHB + (wid < NXTRA).astype(_i32)

    @pl.when(wid < NXTRA)
    def _():
        pltpu.sync_copy(src_hbm.at[pl.ds(base, NCHB + 1)], srcv)
        pltpu.sync_copy(dst_hbm.at[pl.ds(base, NCHB + 1)], dstv)

    @pl.when(wid >= NXTRA)
    def _():
        pltpu.sync_copy(src_hbm.at[pl.ds(base, NCHB)], srcv.at[pl.ds(0, NCHB)])
        pltpu.sync_copy(dst_hbm.at[pl.ds(base, NCHB)], dstv.at[pl.ds(0, NCHB)])

    plsc.subcore_barrier()
    _agg_stream(u_hbm, srcv, dstv, bufs, semg, sems, shared, nch)
    plsc.subcore_barrier()
    pltpu.sync_copy(shared.at[pl.ds(zrow, ZPT)], out_hbm.at[c, pl.ds(zrow, ZPT)])


@functools.cache
def _get_sc_kernels():
    mesh = plsc.VectorSubcoreMesh(core_axis_name="c", subcore_axis_name="s",
                                  num_cores=NSC, num_subcores=NTPC)
    params = pltpu.CompilerParams(use_tc_tiling_on_sc=False,
                                  needs_layout_passes=False)
    s01 = pl.kernel(
        _s01_body,
        out_type=(jax.ShapeDtypeStruct((NR, F), _f32),
                  jax.ShapeDtypeStruct((NR, F), _f32),
                  jax.ShapeDtypeStruct((NR, F), _f32),
                  jax.ShapeDtypeStruct((NR, F), _f32),
                  jax.ShapeDtypeStruct((NSC, NR, F), _f32)),
        mesh=mesh,
        scratch_types=[
            pltpu.VMEM((NCHT // NTPC + 1, CHUNK), _i32),
            pltpu.VMEM((NR,), _i32),
            pltpu.VMEM((NTPC, ZPT), _i32),
            pltpu.VMEM((5 * ZPT,), _f32),
            pltpu.VMEM((ZPT, F), _f32),
            pltpu.VMEM((ZPT, F), _f32),
            pltpu.VMEM((ZPT,), _f32),
            pltpu.VMEM((NCHB + 1, CHUNK), _i32),
            pltpu.VMEM((NCHB + 1, CHUNK), _i32),
            pltpu.VMEM((RING, CHUNK, F), _f32),
            pltpu.SemaphoreType.DMA,
            pltpu.SemaphoreType.DMA,
            pltpu.VMEM_SHARED((NTPC, NR), _i32),
            pltpu.VMEM_SHARED((NR, F), _f32),
        ],
        compiler_params=params,
    )
    agg = pl.kernel(
        _agg_body,
        out_type=jax.ShapeDtypeStruct((NSC, NR, F), _f32),
        mesh=mesh,
        scratch_types=[
            pltpu.VMEM((NCHB + 1, CHUNK), _i32),
            pltpu.VMEM((NCHB + 1, CHUNK), _i32),
            pltpu.VMEM((RING, CHUNK, F), _f32),
            pltpu.SemaphoreType.DMA,
            pltpu.SemaphoreType.DMA,
            pltpu.VMEM_SHARED((NR, F), _f32),
        ],
        compiler_params=params,
    )
    return s01, agg


BT2 = 2560


def _t2_body(da_ref, db_ref, u1_ref, dis_ref, w1_ref, b1_ref, w2_ref, u2_ref):
    y = dis_ref[...] * (da_ref[0] + db_ref[0] + u1_ref[...])
    h = jnp.dot(y, w1_ref[...], preferred_element_type=_f32) + b1_ref[...]
    h = jnp.maximum(h, 0.0)
    z = jnp.dot(h, w2_ref[...], preferred_element_type=_f32)
    u2_ref[...] = dis_ref[...] * z


_t2 = pl.pallas_call(
    _t2_body,
    grid=(NR // BT2,),
    in_specs=[
        pl.BlockSpec((1, BT2, F), lambda i: (0, i, 0)),
        pl.BlockSpec((1, BT2, F), lambda i: (1, i, 0)),
        pl.BlockSpec((BT2, F), lambda i: (i, 0)),
        pl.BlockSpec((BT2, F), lambda i: (i, 0)),
        pl.BlockSpec((F, 1024), lambda i: (0, 0)),
        pl.BlockSpec((1, 1024), lambda i: (0, 0)),
        pl.BlockSpec((1024, F), lambda i: (0, 0)),
    ],
    out_specs=pl.BlockSpec((BT2, F), lambda i: (i, 0)),
    out_shape=jax.ShapeDtypeStruct((NR, F), _f32),
)


def _t3_body(da_ref, db_ref, u2_ref, dis_ref, b2_ref, out_ref):
    out_ref[...] = dis_ref[...] * (da_ref[...] + db_ref[...] + u2_ref[...]) + b2_ref[...]


_t3 = pl.pallas_call(
    _t3_body,
    grid=(1,),
    in_specs=[
        pl.BlockSpec((NRR, 128), lambda i: (0, 0)),
        pl.BlockSpec((NRR, 128), lambda i: (1, 0)),
        pl.BlockSpec((NRR, 128), lambda i: (0, 0)),
        pl.BlockSpec((NRR, 128), lambda i: (0, 0)),
        pl.BlockSpec((1, 128), lambda i: (0, 0)),
    ],
    out_specs=pl.BlockSpec((NRR, 128), lambda i: (0, 0)),
    out_shape=jax.ShapeDtypeStruct((NRR, 128), _f32),
)


def kernel(e, pt, eta, phi, m, edge_index, W1, b1, W2, b2):
    ei = edge_index.astype(_i32)
    src2d = ei[0].reshape(NCHT, CHUNK)
    dst2d = ei[1].reshape(NCHT, CHUNK)
    zeros = jnp.zeros((NR, F), _f32)
    W1p = jnp.zeros((F, 1024), _f32).at[:5].set(W1.astype(_f32))
    W2p = jnp.zeros((1024, F), _f32).at[:, :3].set(W2.astype(_f32))
    b2t = jnp.tile(jnp.zeros((F,), _f32).at[:3].set(b2.astype(_f32)), 128 // F)[None]

    s01, agg = _get_sc_kernels()
    u1a, u1b, disa, disb, d1 = s01(
        e.astype(_f32).reshape(-1), pt.astype(_f32).reshape(-1),
        eta.astype(_f32).reshape(-1), phi.astype(_f32).reshape(-1),
        m.astype(_f32).reshape(-1), src2d, dst2d, zeros)
    u2 = _t2(d1, d1, u1a, disa, W1p, b1.astype(_f32)[None], W2p)
    d2 = agg(u2, src2d, dst2d, zeros)
    d2v = d2.reshape(2 * NRR, 128)
    outv = _t3(d2v, d2v, u2.reshape(NRR, 128), disa.reshape(NRR, 128), b2t)
    return outv.reshape(NR, F)[:N, :3]


# speedup vs baseline: 1.0485x; 1.0485x over previous
import functools

import jax
import jax.numpy as jnp
from jax import lax
from jax.experimental import pallas as pl
from jax.experimental.pallas import tpu as pltpu
from jax.experimental.pallas import tpu_sc as plsc

N = 10000
E = 160000
NR = 10240
F = 8
NSC = 2
NTPC = 16
NTILES = NSC * NTPC
CHUNK = 128
EPT0 = E // NTPC
NCHT = E // CHUNK
NCHB = NCHT // NTILES
NXTRA = NCHT - NCHB * NTILES
HALF = NR // NSC
RPT = HALF // NTPC
ZPT = NR // NTPC
NRR = NR * F // 128
RING = 16
LOOK = 8
SQ = 8

_f32 = jnp.float32
_i32 = jnp.int32


def _rsqrt_newton(d):
    i = plsc.bitcast(d, _i32)
    y = plsc.bitcast(0x5F3759DF - (i >> 1), _f32)
    y = y * (1.5 - 0.5 * d * y * y)
    y = y * (1.5 - 0.5 * d * y * y)
    y = y * (1.5 - 0.5 * d * y * y)
    return y


def _s0_body(e_hbm, pt_hbm, eta_hbm, phi_hbm, m_hbm, dst_hbm, u1_hbm, dis_hbm,
             dstv, histv, hpart, colb, ubf, disbf, disloc, hists):
    c = lax.axis_index("c")
    s = lax.axis_index("s")

    @pl.loop(0, NR // 16)
    def _zero(j):
        histv[pl.ds(j * 16, 16)] = jnp.zeros((16,), _i32)

    pltpu.sync_copy(dst_hbm.at[pl.ds(s * EPT0, EPT0)], dstv)

    @pl.loop(0, EPT0 // 16, unroll=4)
    def _count(j):
        dv = dstv[pl.ds(j * 16, 16)]
        cnt, last = plsc.scan_count(dv)
        plsc.addupdate_scatter(histv, [dv], cnt, mask=last)

    pltpu.sync_copy(histv, hists.at[s])

    @pl.loop(0, 5 * RPT // 16)
    def _zc(j):
        colb[pl.ds(j * 16, 16)] = jnp.zeros((16,), _f32)

    base = c * HALF + s * RPT
    feats = (e_hbm, pt_hbm, eta_hbm, phi_hbm, m_hbm)

    @pl.when(base + RPT <= N)
    def _():
        for f, fh in enumerate(feats):
            pltpu.sync_copy(fh.at[pl.ds(base, RPT)], colb.at[pl.ds(f * RPT, RPT)])

    @pl.when(base + RPT > N)
    def _():
        for f, fh in enumerate(feats):
            pltpu.sync_copy(fh.at[pl.ds(base, N - HALF - 15 * RPT)],
                            colb.at[pl.ds(f * RPT, N - HALF - 15 * RPT)])

    plsc.subcore_barrier()

    for t in range(NTPC):
        pltpu.sync_copy(hists.at[t, pl.ds(base, RPT)], hpart.at[t])

    @pl.loop(0, RPT // 16)
    def _deg(j):
        tot = hpart[0, pl.ds(j * 16, 16)]
        for t in range(1, NTPC):
            tot = tot + hpart[t, pl.ds(j * 16, 16)]
        d = tot.astype(_f32) + 1.0
        disloc[pl.ds(j * 16, 16)] = _rsqrt_newton(d)

    iota = lax.iota(_i32, 16)

    @pl.loop(0, RPT * F // 16, unroll=2)
    def _scale(j):
        g = 16 * j + iota
        r = g >> 3
        f = g & 7
        xv = plsc.load_gather(colb, [jnp.minimum(f * RPT + r, 5 * RPT - 1)])
        xv = jnp.where(f < 5, xv, 0.0)
        dv = plsc.load_gather(disloc, [r])
        disbf[pl.ds(16 * j, 16)] = dv
        ubf[pl.ds(16 * j, 16)] = dv * xv

    pltpu.sync_copy(disbf, dis_hbm.at[pl.ds(base * F, RPT * F)])
    pltpu.sync_copy(ubf, u1_hbm.at[pl.ds(base * F, RPT * F)])


def _agg_body(u_hbm, src_hbm, dst_hbm, zeros_hbm, out_hbm,
              srcv, dstv, bufs, semg, sems, shared):
    c = lax.axis_index("c")
    s = lax.axis_index("s")
    wid = c * NTPC + s
    zrow = s * ZPT
    pltpu.sync_copy(zeros_hbm.at[pl.ds(zrow, ZPT)], shared.at[pl.ds(zrow, ZPT)])

    base = wid * NCHB + jnp.minimum(wid, NXTRA)
    nch = NCHB + (wid < NXTRA).astype(_i32)

    @pl.when(wid < NXTRA)
    def _():
        pltpu.sync_copy(src_hbm.at[pl.ds(base, NCHB + 1)], srcv)
        pltpu.sync_copy(dst_hbm.at[pl.ds(base, NCHB + 1)], dstv)

    @pl.when(wid >= NXTRA)
    def _():
        pltpu.sync_copy(src_hbm.at[pl.ds(base, NCHB)], srcv.at[pl.ds(0, NCHB)])
        pltpu.sync_copy(dst_hbm.at[pl.ds(base, NCHB)], dstv.at[pl.ds(0, NCHB)])

    plsc.subcore_barrier()

    @pl.loop(0, LOOK)
    def _prime(j):
        pltpu.async_copy(u_hbm.at[srcv.at[j]], bufs.at[j], semg)

    @pl.loop(0, nch)
    def _main(j):
        @pl.when(j + LOOK < nch)
        def _():
            jn = j + LOOK
            b = lax.rem(jn, RING)

            @pl.when(jn >= RING)
            def _():
                pltpu.make_async_copy(bufs.at[b], shared.at[dstv.at[jn - RING]],
                                      sems).wait()

            pltpu.async_copy(u_hbm.at[srcv.at[jn]], bufs.at[b], semg)

        bj = lax.rem(j, RING)
        pltpu.make_async_copy(u_hbm.at[srcv.at[j]], bufs.at[bj], semg).wait()
        pltpu.async_copy(bufs.at[bj], shared.at[dstv.at[j]], sems, add=True)

    @pl.loop(nch - RING, nch)
    def _drain(j):
        pltpu.make_async_copy(bufs.at[lax.rem(j, RING)], shared.at[dstv.at[j]],
                              sems).wait()

    plsc.subcore_barrier()
    pltpu.sync_copy(shared.at[pl.ds(zrow, ZPT)], out_hbm.at[c, pl.ds(zrow, ZPT)])


@functools.cache
def _get_sc_kernels():
    mesh = plsc.VectorSubcoreMesh(core_axis_name="c", subcore_axis_name="s",
                                  num_cores=NSC, num_subcores=NTPC)
    params = pltpu.CompilerParams(use_tc_tiling_on_sc=False,
                                  needs_layout_passes=False)
    s0 = pl.kernel(
        _s0_body,
        out_type=(jax.ShapeDtypeStruct((NR * F,), _f32),
                  jax.ShapeDtypeStruct((NR * F,), _f32)),
        mesh=mesh,
        scratch_types=[
            pltpu.VMEM((EPT0,), _i32),
            pltpu.VMEM((NR,), _i32),
            pltpu.VMEM((NTPC, RPT), _i32),
            pltpu.VMEM((5 * RPT,), _f32),
            pltpu.VMEM((RPT * F,), _f32),
            pltpu.VMEM((RPT * F,), _f32),
            pltpu.VMEM((RPT,), _f32),
            pltpu.VMEM_SHARED((NTPC, NR), _i32),
        ],
        compiler_params=params,
    )
    agg = pl.kernel(
        _agg_body,
        out_type=jax.ShapeDtypeStruct((NSC, NR, F), _f32),
        mesh=mesh,
        scratch_types=[
            pltpu.VMEM((NCHB + 1, CHUNK), _i32),
            pltpu.VMEM((NCHB + 1, CHUNK), _i32),
            pltpu.VMEM((RING, CHUNK, F), _f32),
            pltpu.SemaphoreType.DMA,
            pltpu.SemaphoreType.DMA,
            pltpu.VMEM_SHARED((NR, F), _f32),
        ],
        compiler_params=params,
    )
    return s0, agg


BT2 = 2560


def _t2_body(da_ref, db_ref, u1_ref, dis_ref, w1_ref, b1_ref, w2_ref, u2_ref):
    y = dis_ref[...] * (da_ref[0] + db_ref[0] + u1_ref[...])
    h = jnp.dot(y, w1_ref[...], preferred_element_type=_f32) + b1_ref[...]
    h = jnp.maximum(h, 0.0)
    z = jnp.dot(h, w2_ref[...], preferred_element_type=_f32)
    u2_ref[...] = dis_ref[...] * z


_t2 = pl.pallas_call(
    _t2_body,
    grid=(NR // BT2,),
    in_specs=[
        pl.BlockSpec((1, BT2, F), lambda i: (0, i, 0)),
        pl.BlockSpec((1, BT2, F), lambda i: (1, i, 0)),
        pl.BlockSpec((BT2, F), lambda i: (i, 0)),
        pl.BlockSpec((BT2, F), lambda i: (i, 0)),
        pl.BlockSpec((F, 1024), lambda i: (0, 0)),
        pl.BlockSpec((1, 1024), lambda i: (0, 0)),
        pl.BlockSpec((1024, F), lambda i: (0, 0)),
    ],
    out_specs=pl.BlockSpec((BT2, F), lambda i: (i, 0)),
    out_shape=jax.ShapeDtypeStruct((NR, F), _f32),
)


def _t3_body(da_ref, db_ref, u2_ref, dis_ref, b2_ref, out_ref):
    out_ref[...] = dis_ref[...] * (da_ref[...] + db_ref[...] + u2_ref[...]) + b2_ref[...]


_t3 = pl.pallas_call(
    _t3_body,
    grid=(1,),
    in_specs=[
        pl.BlockSpec((NRR, 128), lambda i: (0, 0)),
        pl.BlockSpec((NRR, 128), lambda i: (1, 0)),
        pl.BlockSpec((NRR, 128), lambda i: (0, 0)),
        pl.BlockSpec((NRR, 128), lambda i: (0, 0)),
        pl.BlockSpec((1, 128), lambda i: (0, 0)),
    ],
    out_specs=pl.BlockSpec((NRR, 128), lambda i: (0, 0)),
    out_shape=jax.ShapeDtypeStruct((NRR, 128), _f32),
)


def kernel(e, pt, eta, phi, m, edge_index, W1, b1, W2, b2):
    ei = edge_index.astype(_i32)
    src2d = ei[0].reshape(NCHT, CHUNK)
    dst2d = ei[1].reshape(NCHT, CHUNK)
    zeros = jnp.zeros((NR, F), _f32)
    W1p = jnp.zeros((F, 1024), _f32).at[:5].set(W1.astype(_f32))
    W2p = jnp.zeros((1024, F), _f32).at[:, :3].set(W2.astype(_f32))
    b2t = jnp.tile(jnp.zeros((F,), _f32).at[:3].set(b2.astype(_f32)), 128 // F)[None]

    s0, agg = _get_sc_kernels()
    u1f, disf = s0(e.astype(_f32).reshape(-1), pt.astype(_f32).reshape(-1),
                   eta.astype(_f32).reshape(-1), phi.astype(_f32).reshape(-1),
                   m.astype(_f32).reshape(-1), ei[1])
    u1 = u1f.reshape(NR, F)
    dis = disf.reshape(NR, F)
    d1 = agg(u1, src2d, dst2d, zeros)
    u2 = _t2(d1, d1, u1, dis, W1p, b1.astype(_f32)[None], W2p)
    d2 = agg(u2, src2d, dst2d, zeros)
    d2v = d2.reshape(2 * NRR, 128)
    outv = _t3(d2v, d2v, u2.reshape(NRR, 128), disf.reshape(NRR, 128), b2t)
    return outv.reshape(NR, F)[:N, :3]
